# Initial kernel scaffold; baseline (speedup 1.0000x reference)
import jax
import jax.numpy as jnp
from jax.experimental import pallas as pl


def kernel(x, table):
    raise NotImplementedError("write your pallas kernel here")


# speedup vs baseline: 2.4896x; 2.4896x over previous
import functools

import jax
import jax.numpy as jnp
from jax import lax
from jax.experimental import pallas as pl
from jax.experimental.pallas import tpu as pltpu
from jax.experimental.pallas import tpu_sc as plsc

VOCAB = 1000000
EMB = 16
BATCH = 16384
HIST = 200
B = BATCH * HIST

NC = 2
NS = 16
NW = NC * NS
BPW = B // NW
CHUNK = 2048
NCHUNK = BPW // CHUNK

_mesh = plsc.VectorSubcoreMesh(core_axis_name="c", subcore_axis_name="s")


@functools.partial(
    pl.kernel,
    mesh=_mesh,
    compiler_params=pltpu.CompilerParams(use_tc_tiling_on_sc=False),
    out_type=jax.ShapeDtypeStruct((B, EMB), jnp.float32),
    scratch_types=[
        pltpu.VMEM((CHUNK,), jnp.int32),
        pltpu.VMEM((CHUNK, EMB), jnp.float32),
        pltpu.SemaphoreType.DMA,
    ],
)
def _gather(idx_hbm, table_hbm, out_hbm, idx_v, rows_v, sem):
    wid = lax.axis_index("s") * NC + lax.axis_index("c")
    base = wid * BPW

    def body(i, carry):
        off = base + i * CHUNK
        pltpu.sync_copy(idx_hbm.at[pl.ds(off, CHUNK)], idx_v)
        pltpu.async_copy(table_hbm.at[idx_v], rows_v, sem).wait()
        pltpu.sync_copy(rows_v, out_hbm.at[pl.ds(off, CHUNK)])
        return carry

    lax.fori_loop(0, NCHUNK, body, 0)


def kernel(x, table):
    flat_idx = x.reshape(B)
    out = _gather(flat_idx, table)
    return out.reshape(BATCH, HIST, EMB)


# speedup vs baseline: 2.5258x; 1.0146x over previous
import functools

import jax
import jax.numpy as jnp
from jax import lax
from jax.experimental import pallas as pl
from jax.experimental.pallas import tpu as pltpu
from jax.experimental.pallas import tpu_sc as plsc

VOCAB = 1000000
EMB = 16
BATCH = 16384
HIST = 200
B = BATCH * HIST

NC = 2
NS = 16
NW = NC * NS
BPW = B // NW
CHUNK = 2048
NPAIR = BPW // (2 * CHUNK)

_mesh = plsc.VectorSubcoreMesh(core_axis_name="c", subcore_axis_name="s")


@functools.partial(
    pl.kernel,
    mesh=_mesh,
    compiler_params=pltpu.CompilerParams(use_tc_tiling_on_sc=False),
    out_type=jax.ShapeDtypeStruct((B, EMB), jnp.float32),
    scratch_types=[
        pltpu.VMEM((CHUNK,), jnp.int32),
        pltpu.VMEM((CHUNK,), jnp.int32),
        pltpu.VMEM((CHUNK, EMB), jnp.float32),
        pltpu.VMEM((CHUNK, EMB), jnp.float32),
        pltpu.SemaphoreType.DMA,
        pltpu.SemaphoreType.DMA,
        pltpu.SemaphoreType.DMA,
        pltpu.SemaphoreType.DMA,
    ],
)
def _gather(idx_hbm, table_hbm, out_hbm, idx0, idx1, rows0, rows1,
            sg0, sg1, sw0, sw1):
    wid = lax.axis_index("s") * NC + lax.axis_index("c")
    base = wid * BPW

    def body(g, carry):
        off0 = base + g * (2 * CHUNK)
        off1 = off0 + CHUNK

        @pl.when(g > 0)
        def _drain():
            pltpu.make_async_copy(rows0, out_hbm.at[pl.ds(off0, CHUNK)], sw0).wait()
            pltpu.make_async_copy(rows1, out_hbm.at[pl.ds(off1, CHUNK)], sw1).wait()

        pltpu.sync_copy(idx_hbm.at[pl.ds(off0, CHUNK)], idx0)
        pltpu.sync_copy(idx_hbm.at[pl.ds(off1, CHUNK)], idx1)
        g0 = pltpu.async_copy(table_hbm.at[idx0], rows0, sg0)
        g1 = pltpu.async_copy(table_hbm.at[idx1], rows1, sg1)
        g0.wait()
        pltpu.async_copy(rows0, out_hbm.at[pl.ds(off0, CHUNK)], sw0)
        g1.wait()
        pltpu.async_copy(rows1, out_hbm.at[pl.ds(off1, CHUNK)], sw1)
        return carry

    lax.fori_loop(0, NPAIR, body, 0)
    pltpu.make_async_copy(rows0, out_hbm.at[pl.ds(base, CHUNK)], sw0).wait()
    pltpu.make_async_copy(rows1, out_hbm.at[pl.ds(base, CHUNK)], sw1).wait()


def kernel(x, table):
    flat_idx = x.reshape(B)
    out = _gather(flat_idx, table)
    return out.reshape(BATCH, HIST, EMB)


# speedup vs baseline: 4.0739x; 1.6129x over previous
import functools

import jax
import jax.numpy as jnp
from jax import lax
from jax.experimental import pallas as pl
from jax.experimental.pallas import tpu as pltpu
from jax.experimental.pallas import tpu_sc as plsc

VOCAB = 1000000
EMB = 16
BATCH = 16384
HIST = 200
B = BATCH * HIST

NC = 2
NS = 16
NW = NC * NS
CHUNK = 1024
CPH = BATCH // CHUNK
NUNIT = HIST * CPH
UPW = NUNIT // NW
NBODY = UPW // 2

_mesh = plsc.VectorSubcoreMesh(core_axis_name="c", subcore_axis_name="s")


@functools.partial(
    pl.kernel,
    mesh=_mesh,
    compiler_params=pltpu.CompilerParams(use_tc_tiling_on_sc=False,
                                         needs_layout_passes=False),
    out_type=jax.ShapeDtypeStruct((B * EMB,), jnp.float32),
    scratch_types=[
        pltpu.VMEM((CHUNK,), jnp.int32),
        pltpu.VMEM((CHUNK,), jnp.int32),
        pltpu.VMEM((CHUNK, EMB), jnp.float32),
        pltpu.VMEM((CHUNK, EMB), jnp.float32),
        pltpu.VMEM((CHUNK * EMB,), jnp.float32),
        pltpu.VMEM((CHUNK * EMB,), jnp.float32),
        pltpu.SemaphoreType.DMA,
        pltpu.SemaphoreType.DMA,
        pltpu.SemaphoreType.DMA,
        pltpu.SemaphoreType.DMA,
    ],
)
def _gather(idx_hbm, table_hbm, out_hbm,
            idxa, idxb, rowsa, rowsb, tbufa, tbufb,
            sga, sgb, swa, swb):
    wid = lax.axis_index("s") * NC + lax.axis_index("c")
    ubase = wid * UPW

    lane = lax.iota(jnp.int32, 16)
    pat = ((lane >> 3) << 13) + ((lane & 7) << 7)

    def idx_off(u):
        return (u // CPH) * (CPH * CHUNK) + (u % CPH) * CHUNK

    def out_off(u):
        return (u // CPH) * (2 * 128 * 1024) + (u % CPH) * (8 * 1024)

    def transpose(rows, tbuf):
        def tc_body(tc, c1):
            def c_body(c, c2):
                j = tc * 128 + c
                row = rows[j]
                plsc.store_scatter(tbuf, [pat + (tc * 1024 + c)], row)
                return c2
            return lax.fori_loop(0, 128, c_body, c1)
        lax.fori_loop(0, 8, tc_body, 0)

    def load_and_fire(u, idxv, rows, sg):
        pltpu.sync_copy(idx_hbm.at[pl.ds(idx_off(u), CHUNK)], idxv)
        pltpu.async_copy(table_hbm.at[idxv], rows, sg)

    def store(u, tbuf, sw):
        o = out_off(u)
        pltpu.async_copy(tbuf.at[pl.ds(0, 8192)],
                         out_hbm.at[pl.ds(o, 8192)], sw)
        pltpu.async_copy(tbuf.at[pl.ds(8192, 8192)],
                         out_hbm.at[pl.ds(o + 128 * 1024, 8192)], sw)

    def drain_store(u, tbuf, sw):
        o = out_off(u)
        pltpu.make_async_copy(tbuf.at[pl.ds(0, 8192)],
                              out_hbm.at[pl.ds(o, 8192)], sw).wait()
        pltpu.make_async_copy(tbuf.at[pl.ds(8192, 8192)],
                              out_hbm.at[pl.ds(o + 128 * 1024, 8192)], sw).wait()

    load_and_fire(ubase + 0, idxa, rowsa, sga)
    load_and_fire(ubase + 1, idxb, rowsb, sgb)

    def body(g, carry):
        ua = ubase + 2 * g
        ub = ua + 1

        pltpu.make_async_copy(table_hbm.at[idxa], rowsa, sga).wait()

        @pl.when(g > 0)
        def _():
            drain_store(ua, tbufa, swa)

        transpose(rowsa, tbufa)
        store(ua, tbufa, swa)

        @pl.when(g < NBODY - 1)
        def _():
            load_and_fire(ua + 2, idxa, rowsa, sga)

        pltpu.make_async_copy(table_hbm.at[idxb], rowsb, sgb).wait()

        @pl.when(g > 0)
        def _():
            drain_store(ub, tbufb, swb)

        transpose(rowsb, tbufb)
        store(ub, tbufb, swb)

        @pl.when(g < NBODY - 1)
        def _():
            load_and_fire(ub + 2, idxb, rowsb, sgb)

        return carry

    lax.fori_loop(0, NBODY, body, 0)
    drain_store(ubase, tbufa, swa)
    drain_store(ubase + 1, tbufb, swb)


def kernel(x, table):
    flat_idx = x.T.reshape(B)
    out = _gather(flat_idx, table)
    out5 = out.reshape(HIST, 2, 128, 8, 128)
    return out5.transpose((2, 4, 0, 1, 3)).reshape(BATCH, HIST, EMB)


# speedup vs baseline: 4.0767x; 1.0007x over previous
import functools

import jax
import jax.numpy as jnp
from jax import lax
from jax.experimental import pallas as pl
from jax.experimental.pallas import tpu as pltpu
from jax.experimental.pallas import tpu_sc as plsc

VOCAB = 1000000
EMB = 16
BATCH = 16384
HIST = 200
B = BATCH * HIST

NC = 2
NS = 16
NW = NC * NS
CHUNK = 1024
CPH = BATCH // CHUNK
NUNIT = HIST * CPH
UPW = NUNIT // NW
NBODY = UPW // 2

_mesh = plsc.VectorSubcoreMesh(core_axis_name="c", subcore_axis_name="s")


@functools.partial(
    pl.kernel,
    mesh=_mesh,
    compiler_params=pltpu.CompilerParams(use_tc_tiling_on_sc=False,
                                         needs_layout_passes=False),
    out_type=jax.ShapeDtypeStruct((B * EMB,), jnp.float32),
    scratch_types=[
        pltpu.VMEM((CHUNK,), jnp.int32),
        pltpu.VMEM((CHUNK,), jnp.int32),
        pltpu.VMEM((CHUNK, EMB), jnp.float32),
        pltpu.VMEM((CHUNK, EMB), jnp.float32),
        pltpu.VMEM((CHUNK * EMB,), jnp.float32),
        pltpu.VMEM((CHUNK * EMB,), jnp.float32),
        pltpu.SemaphoreType.DMA,
        pltpu.SemaphoreType.DMA,
        pltpu.SemaphoreType.DMA,
        pltpu.SemaphoreType.DMA,
    ],
)
def _gather(idx_hbm, table_hbm, out_hbm,
            idxa, idxb, rowsa, rowsb, tbufa, tbufb,
            sga, sgb, swa, swb):
    wid = lax.axis_index("s") * NC + lax.axis_index("c")
    ubase = wid * UPW

    lane = lax.iota(jnp.int32, 16)
    pat = ((lane >> 3) << 13) + ((lane & 7) << 7)

    def idx_off(u):
        return (u // CPH) * (CPH * CHUNK) + (u % CPH) * CHUNK

    def out_off(u):
        return (u // CPH) * (2 * 128 * 1024) + (u % CPH) * (8 * 1024)

    def transpose(rows, tbuf):
        def tc_body(tc, c1):
            def c_body(c, c2):
                j = tc * 128 + c
                row = rows[j]
                plsc.store_scatter(tbuf, [pat + (tc * 1024 + c)], row)
                return c2
            return lax.fori_loop(0, 128, c_body, c1, unroll=16)
        lax.fori_loop(0, 8, tc_body, 0)

    def load_and_fire(u, idxv, rows, sg):
        pltpu.sync_copy(idx_hbm.at[pl.ds(idx_off(u), CHUNK)], idxv)
        pltpu.async_copy(table_hbm.at[idxv], rows, sg)

    def store(u, tbuf, sw):
        o = out_off(u)
        pltpu.async_copy(tbuf.at[pl.ds(0, 8192)],
                         out_hbm.at[pl.ds(o, 8192)], sw)
        pltpu.async_copy(tbuf.at[pl.ds(8192, 8192)],
                         out_hbm.at[pl.ds(o + 128 * 1024, 8192)], sw)

    def drain_store(u, tbuf, sw):
        o = out_off(u)
        pltpu.make_async_copy(tbuf.at[pl.ds(0, 8192)],
                              out_hbm.at[pl.ds(o, 8192)], sw).wait()
        pltpu.make_async_copy(tbuf.at[pl.ds(8192, 8192)],
                              out_hbm.at[pl.ds(o + 128 * 1024, 8192)], sw).wait()

    load_and_fire(ubase + 0, idxa, rowsa, sga)
    load_and_fire(ubase + 1, idxb, rowsb, sgb)

    def body(g, carry):
        ua = ubase + 2 * g
        ub = ua + 1

        pltpu.make_async_copy(table_hbm.at[idxa], rowsa, sga).wait()

        @pl.when(g > 0)
        def _():
            drain_store(ua, tbufa, swa)

        transpose(rowsa, tbufa)
        store(ua, tbufa, swa)

        @pl.when(g < NBODY - 1)
        def _():
            load_and_fire(ua + 2, idxa, rowsa, sga)

        pltpu.make_async_copy(table_hbm.at[idxb], rowsb, sgb).wait()

        @pl.when(g > 0)
        def _():
            drain_store(ub, tbufb, swb)

        transpose(rowsb, tbufb)
        store(ub, tbufb, swb)

        @pl.when(g < NBODY - 1)
        def _():
            load_and_fire(ub + 2, idxb, rowsb, sgb)

        return carry

    lax.fori_loop(0, NBODY, body, 0)
    drain_store(ubase, tbufa, swa)
    drain_store(ubase + 1, tbufb, swb)


def kernel(x, table):
    flat_idx = x.T.reshape(B)
    out = _gather(flat_idx, table)
    out5 = out.reshape(HIST, 2, 128, 8, 128)
    return out5.transpose((2, 4, 0, 1, 3)).reshape(BATCH, HIST, EMB)


# speedup vs baseline: 7.0345x; 1.7255x over previous
import functools

import jax
import jax.numpy as jnp
from jax import lax
from jax.experimental import pallas as pl
from jax.experimental.pallas import tpu as pltpu
from jax.experimental.pallas import tpu_sc as plsc

VOCAB = 1000000
EMB = 16
BATCH = 16384
HIST = 200
B = BATCH * HIST

NC = 2
NS = 16
NW = NC * NS
CHUNK = 1024
CPH = BATCH // CHUNK
NUNIT = HIST * CPH
UPW = NUNIT // NW
NBODY = UPW // 2

_mesh = plsc.VectorSubcoreMesh(core_axis_name="c", subcore_axis_name="s")


@functools.partial(
    pl.kernel,
    mesh=_mesh,
    compiler_params=pltpu.CompilerParams(use_tc_tiling_on_sc=False,
                                         needs_layout_passes=False),
    out_type=jax.ShapeDtypeStruct((B * EMB,), jnp.float32),
    scratch_types=[
        pltpu.VMEM((CHUNK,), jnp.int32),
        pltpu.VMEM((CHUNK,), jnp.int32),
        pltpu.VMEM((CHUNK, EMB), jnp.float32),
        pltpu.VMEM((CHUNK, EMB), jnp.float32),
        pltpu.VMEM((CHUNK * EMB,), jnp.float32),
        pltpu.VMEM((CHUNK * EMB,), jnp.float32),
        pltpu.SemaphoreType.DMA,
        pltpu.SemaphoreType.DMA,
        pltpu.SemaphoreType.DMA,
        pltpu.SemaphoreType.DMA,
        pltpu.SemaphoreType.DMA,
        pltpu.SemaphoreType.DMA,
    ],
)
def _gather(idx_hbm, table_hbm, out_hbm,
            idxa, idxb, rowsa, rowsb, tbufa, tbufb,
            sga, sgb, swa, swb, sia, sib):
    wid = lax.axis_index("s") * NC + lax.axis_index("c")
    ubase = wid * UPW

    lane = lax.iota(jnp.int32, 16)
    cols = [(lane + d0) & 15 for d0 in range(16)]
    dsts = [((c >> 3) << 13) + ((c & 7) << 7) + lane for c in cols]

    def idx_off(u):
        return (u // CPH) * (CPH * CHUNK) + (u % CPH) * CHUNK

    def out_off(u):
        return (u // CPH) * (2 * 128 * 1024) + (u % CPH) * (8 * 1024)

    def transpose(rows, tbuf):
        def c0_body(blk, carry):
            c0 = blk * 16
            row0 = lane + c0
            ds = ((c0 >> 7) << 10) + (c0 & 127)
            for d0 in range(16):
                v = plsc.load_gather(rows, [row0, cols[d0]])
                plsc.store_scatter(tbuf, [dsts[d0] + ds], v)
            return carry
        lax.fori_loop(0, CHUNK // 16, c0_body, 0)

    def fire_idx(u, idxv, si):
        pltpu.async_copy(idx_hbm.at[pl.ds(idx_off(u), CHUNK)], idxv, si)

    def fire_gather(u, idxv, rows, sg, si):
        pltpu.make_async_copy(idx_hbm.at[pl.ds(idx_off(u), CHUNK)],
                              idxv, si).wait()
        pltpu.async_copy(table_hbm.at[idxv], rows, sg)

    def store(u, tbuf, sw):
        o = out_off(u)
        pltpu.async_copy(tbuf.at[pl.ds(0, 8192)],
                         out_hbm.at[pl.ds(o, 8192)], sw)
        pltpu.async_copy(tbuf.at[pl.ds(8192, 8192)],
                         out_hbm.at[pl.ds(o + 128 * 1024, 8192)], sw)

    def drain_store(u, tbuf, sw):
        o = out_off(u)
        pltpu.make_async_copy(tbuf.at[pl.ds(0, 8192)],
                              out_hbm.at[pl.ds(o, 8192)], sw).wait()
        pltpu.make_async_copy(tbuf.at[pl.ds(8192, 8192)],
                              out_hbm.at[pl.ds(o + 128 * 1024, 8192)], sw).wait()

    fire_idx(ubase + 0, idxa, sia)
    fire_idx(ubase + 1, idxb, sib)
    fire_gather(ubase + 0, idxa, rowsa, sga, sia)
    fire_gather(ubase + 1, idxb, rowsb, sgb, sib)

    def body(g, carry):
        ua = ubase + 2 * g
        ub = ua + 1

        pltpu.make_async_copy(table_hbm.at[idxa], rowsa, sga).wait()

        @pl.when(g < NBODY - 1)
        def _():
            fire_idx(ua + 2, idxa, sia)

        @pl.when(g > 0)
        def _():
            drain_store(ua, tbufa, swa)

        transpose(rowsa, tbufa)
        store(ua, tbufa, swa)

        @pl.when(g < NBODY - 1)
        def _():
            fire_gather(ua + 2, idxa, rowsa, sga, sia)

        pltpu.make_async_copy(table_hbm.at[idxb], rowsb, sgb).wait()

        @pl.when(g < NBODY - 1)
        def _():
            fire_idx(ub + 2, idxb, sib)

        @pl.when(g > 0)
        def _():
            drain_store(ub, tbufb, swb)

        transpose(rowsb, tbufb)
        store(ub, tbufb, swb)

        @pl.when(g < NBODY - 1)
        def _():
            fire_gather(ub + 2, idxb, rowsb, sgb, sib)

        return carry

    lax.fori_loop(0, NBODY, body, 0)
    drain_store(ubase, tbufa, swa)
    drain_store(ubase + 1, tbufb, swb)


def kernel(x, table):
    flat_idx = x.T.reshape(B)
    out = _gather(flat_idx, table)
    out5 = out.reshape(HIST, 2, 128, 8, 128)
    return out5.transpose((2, 4, 0, 1, 3)).reshape(BATCH, HIST, EMB)
